# Initial kernel scaffold; baseline (speedup 1.0000x reference)
import jax
import jax.numpy as jnp
from jax.experimental import pallas as pl


def kernel(x, group_by, W, b):
    raise NotImplementedError("write your pallas kernel here")


# speedup vs baseline: 1.7570x; 1.7570x over previous
import functools

import jax
import jax.numpy as jnp
from jax.experimental import pallas as pl
from jax.experimental.pallas import tpu as pltpu

DIM_IN = 1024
DIM_OUT = 1024
NUM_GROUPS = 8
TOKENS = 8192
TOK_BLK = 512


def _body(rank_ref, g_ref, x_ref, w_ref, b_ref, o_ref):
    j = pl.program_id(1)
    y = jax.lax.dot_general(
        x_ref[...], w_ref[0],
        (((1,), (1,)), ((), ())),
        preferred_element_type=jnp.float32,
    ) + b_ref[0, 0][None, :]
    mask = g_ref[0] == j

    @pl.when(j == 0)
    def _():
        o_ref[...] = jnp.where(mask, y, 0.0)

    @pl.when(j > 0)
    def _():
        o_ref[...] = jnp.where(mask, y, o_ref[...])


@jax.jit
def kernel(x, group_by, W, b):
    g = group_by.astype(jnp.int32)
    counts = jnp.zeros((NUM_GROUPS,), jnp.int32).at[g].add(1)
    rank = jnp.cumsum((counts > 0).astype(jnp.int32)) - 1

    W3 = W.reshape(NUM_GROUPS, DIM_OUT, DIM_IN)
    nb = TOKENS // TOK_BLK
    g3 = g.reshape(nb, TOK_BLK, 1)
    b3 = b.reshape(NUM_GROUPS, 1, DIM_OUT)

    grid_spec = pltpu.PrefetchScalarGridSpec(
        num_scalar_prefetch=1,
        grid=(nb, NUM_GROUPS),
        in_specs=[
            pl.BlockSpec((1, TOK_BLK, 1), lambda i, j, r: (i, 0, 0)),
            pl.BlockSpec((TOK_BLK, DIM_IN), lambda i, j, r: (i, 0)),
            pl.BlockSpec((1, DIM_OUT, DIM_IN), lambda i, j, r: (r[j], 0, 0)),
            pl.BlockSpec((1, 1, DIM_OUT), lambda i, j, r: (r[j], 0, 0)),
        ],
        out_specs=pl.BlockSpec((TOK_BLK, DIM_OUT), lambda i, j, r: (i, 0)),
    )
    out = pl.pallas_call(
        _body,
        grid_spec=grid_spec,
        out_shape=jax.ShapeDtypeStruct((TOKENS, DIM_OUT), jnp.float32),
        compiler_params=pltpu.CompilerParams(
            dimension_semantics=("parallel", "arbitrary"),
        ),
    )(rank, g3, x, W3, b3)
    return out


# speedup vs baseline: 1.8292x; 1.0411x over previous
import functools

import jax
import jax.numpy as jnp
from jax import lax
from jax.experimental import pallas as pl
from jax.experimental.pallas import tpu as pltpu
from jax.experimental.pallas import tpu_sc as plsc

DIM_IN = 1024
DIM_OUT = 1024
NUM_GROUPS = 8
TOKENS = 8192
BLK = 256
NB = TOKENS // BLK + NUM_GROUPS - 1
P = NB * BLK


@functools.cache
def _make_sc_gather(n_src, n_out, chunk):
    info = plsc.get_sparse_core_info()
    nc, ns = info.num_cores, info.num_subcores
    nw = nc * ns
    rows_pw = n_out // nw
    n_chunks = rows_pw // chunk
    assert rows_pw % chunk == 0 and chunk % 8 == 0 and chunk <= 128

    def body(src_hbm, idx_hbm, out_hbm, idx0, idx1, buf0, buf1, sem0, sem1):
        wid = lax.axis_index("s") * nc + lax.axis_index("c")
        base = wid * rows_pw
        idxs = (idx0, idx1)
        bufs = (buf0, buf1)
        sems = (sem0, sem1)
        pltpu.sync_copy(idx_hbm.at[pl.ds(base, chunk)], idx0)
        handles = [None, None]
        handles[0] = pltpu.async_copy(src_hbm.at[idx0], buf0, sem0)
        for c in range(n_chunks):
            i = c % 2
            if c + 1 < n_chunks:
                j = (c + 1) % 2
                pltpu.sync_copy(
                    idx_hbm.at[pl.ds(base + (c + 1) * chunk, chunk)], idxs[j])
                handles[j] = pltpu.async_copy(src_hbm.at[idxs[j]], bufs[j], sems[j])
            handles[i].wait()
            pltpu.sync_copy(bufs[i], out_hbm.at[pl.ds(base + c * chunk, chunk)])

    return pl.kernel(
        body,
        out_type=jax.ShapeDtypeStruct((n_out, DIM_IN), jnp.float32),
        mesh=plsc.VectorSubcoreMesh(core_axis_name="c", subcore_axis_name="s"),
        scratch_types=[
            pltpu.VMEM((chunk,), jnp.int32),
            pltpu.VMEM((chunk,), jnp.int32),
            pltpu.VMEM((chunk, DIM_IN), jnp.float32),
            pltpu.VMEM((chunk, DIM_IN), jnp.float32),
            pltpu.SemaphoreType.DMA,
            pltpu.SemaphoreType.DMA,
        ],
    )


def _mm_body(widx_ref, x_ref, w_ref, b_ref, o_ref):
    o_ref[...] = jax.lax.dot_general(
        x_ref[...], w_ref[0],
        (((1,), (1,)), ((), ())),
        preferred_element_type=jnp.float32,
    ) + b_ref[0, 0][None, :]


def _tc_group_matmul(w_idx, xs, W3, b3):
    grid_spec = pltpu.PrefetchScalarGridSpec(
        num_scalar_prefetch=1,
        grid=(NB,),
        in_specs=[
            pl.BlockSpec((BLK, DIM_IN), lambda i, r: (i, 0)),
            pl.BlockSpec((1, DIM_OUT, DIM_IN), lambda i, r: (r[i], 0, 0)),
            pl.BlockSpec((1, 1, DIM_OUT), lambda i, r: (r[i], 0, 0)),
        ],
        out_specs=pl.BlockSpec((BLK, DIM_OUT), lambda i, r: (i, 0)),
    )
    return pl.pallas_call(
        _mm_body,
        grid_spec=grid_spec,
        out_shape=jax.ShapeDtypeStruct((P, DIM_OUT), jnp.float32),
        compiler_params=pltpu.CompilerParams(
            dimension_semantics=("parallel",),
        ),
    )(w_idx, xs, W3, b3)


def _route(g):
    onehot = (g[:, None] == jnp.arange(NUM_GROUPS, dtype=jnp.int32)[None, :])
    cum = jnp.cumsum(onehot.astype(jnp.int32), axis=0)
    counts = cum[-1]
    r_t = jnp.take_along_axis(cum, g[:, None], axis=1)[:, 0] - 1
    present = counts > 0
    rank = jnp.cumsum(present.astype(jnp.int32)) - 1
    padded = ((counts + BLK - 1) // BLK) * BLK
    pad_end = jnp.cumsum(padded)
    pad_start = pad_end - padded
    pos = pad_start[g] + r_t
    gather_idx = jnp.zeros((P,), jnp.int32).at[pos].set(
        jnp.arange(TOKENS, dtype=jnp.int32), unique_indices=True)
    blk_lo = jnp.arange(NB, dtype=jnp.int32) * BLK
    blk_gid = jnp.minimum(
        jnp.searchsorted(pad_end, blk_lo, side="right").astype(jnp.int32),
        NUM_GROUPS - 1)
    w_idx = jnp.maximum(rank[blk_gid], 0)
    return gather_idx, w_idx, pos


@jax.jit
def kernel(x, group_by, W, b):
    g = group_by.astype(jnp.int32)
    gather_idx, w_idx, pos = _route(g)

    W3 = W.reshape(NUM_GROUPS, DIM_OUT, DIM_IN)
    b3 = b.reshape(NUM_GROUPS, 1, DIM_OUT)

    xs = _make_sc_gather(TOKENS, P, 24)(x, gather_idx)
    ys = _tc_group_matmul(w_idx, xs, W3, b3)
    out = _make_sc_gather(P, TOKENS, 32)(ys, pos)
    return out
